# baseline (speedup 1.0000x reference)
import functools

import jax
import jax.numpy as jnp
from jax import lax
from jax.experimental import pallas as pl
from jax.experimental.pallas import tpu as pltpu
from jax.experimental.pallas import tpu_sc as plsc

_NUM_BUCKETS = 32
_NUM_HEADS = 16
_SEQ = 2048
_LANES = 16
_LCOLS = 4224
_NT = 2 * (_LCOLS // 128)
_BQ = 128
_W = _SEQ + _BQ
_T = (1, 2, 3, 4, 5, 6, 7, 8, 12, 16, 23, 32, 46, 64, 91)


def _perm16(x, idx):
    dnums = lax.GatherDimensionNumbers(
        offset_dims=(), collapsed_slice_dims=(0,), start_index_map=(0,))
    return lax.gather(x, idx[:, None], dnums, (1,),
                      mode=lax.GatherScatterMode.PROMISE_IN_BOUNDS)


def _line_body(tablet_hbm, line_hbm, tbl_v, stage_v, sem):
    wid = lax.axis_index("s") * 2 + lax.axis_index("c")
    pltpu.sync_copy(tablet_hbm, tbl_v)
    iota = lax.iota(jnp.int32, _LANES)

    def do_tile(tile):
        g = tile // (_NT // 2)
        t = tile % (_NT // 2)
        for p in range(8):
            hh = g * 8 + p
            off = pl.multiple_of(hh * _NUM_BUCKETS, _LANES)
            lo = tbl_v[pl.ds(off, _LANES)]
            hi = tbl_v[pl.ds(off + _LANES, _LANES)]
            for cc in range(8):
                j = t * 128 + cc * _LANES + iota
                d = j - (_SEQ - 1)
                n = jnp.abs(d)
                b = jnp.full((_LANES,), 0, jnp.int32)
                for th in _T:
                    b = b + jnp.where(n >= th, 1, 0).astype(jnp.int32)
                b = b + jnp.where(d > 0, _NUM_BUCKETS // 2, 0)
                v = jnp.where(b < _LANES,
                              _perm16(lo, jnp.bitwise_and(b, _LANES - 1)),
                              _perm16(hi, jnp.bitwise_and(b, _LANES - 1)))
                stage_v[p, pl.ds(cc * _LANES, _LANES)] = v
        row0 = pl.multiple_of(g * 8, 8)
        col0 = pl.multiple_of(t * 128, 128)
        pltpu.async_copy(
            stage_v,
            line_hbm.at[pl.ds(row0, 8), pl.ds(col0, 128)], sem).wait()

    for rep in range(3):
        tile = wid + 32 * rep
        if rep < 2:
            do_tile(tile)
        else:
            @pl.when(tile < _NT)
            def _():
                do_tile(tile)


def _bias_body(line_ref, out_ref):
    qi = pl.program_id(0)
    base = pl.multiple_of((pl.num_programs(0) - 1 - qi) * _BQ, _BQ)
    for h in range(_NUM_HEADS):
        w = line_ref[h, pl.ds(base, _W)]
        x = jax.lax.broadcast_in_dim(w, (_BQ, _W), (1,))
        rolled = pltpu.roll(x, _W - (_BQ - 1), 1, stride=1, stride_axis=0)
        out_ref[0, h, :, :] = rolled[:, :_SEQ]


def kernel(q_len, k_len, bidirectional, table):
    del q_len, k_len, bidirectional
    mesh = plsc.VectorSubcoreMesh(core_axis_name="c", subcore_axis_name="s")
    line = functools.partial(
        pl.kernel,
        mesh=mesh,
        out_type=jax.ShapeDtypeStruct((_NUM_HEADS, _LCOLS), jnp.float32),
        scratch_types=[
            pltpu.VMEM((_NUM_HEADS * _NUM_BUCKETS,), jnp.float32),
            pltpu.VMEM((8, 128), jnp.float32),
            pltpu.SemaphoreType.DMA,
        ],
    )(_line_body)(table.T.reshape(_NUM_HEADS * _NUM_BUCKETS))
    return pl.pallas_call(
        _bias_body,
        grid=(_SEQ // _BQ,),
        in_specs=[pl.BlockSpec((_NUM_HEADS, _LCOLS), lambda i: (0, 0))],
        out_specs=pl.BlockSpec(
            (1, _NUM_HEADS, _BQ, _SEQ), lambda i: (0, 0, i, 0)),
        out_shape=jax.ShapeDtypeStruct(
            (1, _NUM_HEADS, _SEQ, _SEQ), jnp.float32),
        compiler_params=pltpu.CompilerParams(
            dimension_semantics=("arbitrary",)),
    )(line)


# speedup vs baseline: 1.3610x; 1.3610x over previous
import math

import jax
import jax.numpy as jnp
from jax.experimental import pallas as pl
from jax.experimental.pallas import tpu as pltpu

_NUM_BUCKETS = 32
_MAX_DISTANCE = 128
_NUM_HEADS = 16
_SEQ = 2048
_LINE = 2 * _SEQ
_BQ = 128
_W = _SEQ + _BQ


def _bias_body(tablet_ref, out_ref, line_ref):
    qi = pl.program_id(0)

    @pl.when(qi == 0)
    def _build_line():
        j = jax.lax.broadcasted_iota(jnp.int32, (_NUM_HEADS, _LINE), 1)
        d = j - (_SEQ - 1)
        n = -d
        half = _NUM_BUCKETS // 2
        sign = (n < 0).astype(jnp.int32)
        n = jnp.abs(n)
        max_exact = half // 2
        is_small = n < max_exact
        val_if_large = max_exact + (
            jnp.log(n.astype(jnp.float32) / max_exact + 1e-06)
            / math.log(_MAX_DISTANCE / max_exact)
            * (half - max_exact)
        ).astype(jnp.int32)
        val_if_large = jnp.minimum(val_if_large, half - 1)
        bucket = jnp.where(is_small, n, val_if_large) + sign * half
        acc = jnp.zeros((_NUM_HEADS, _LINE), jnp.float32)
        for b in range(_NUM_BUCKETS):
            col = jax.lax.broadcast_in_dim(
                tablet_ref[:, b:b + 1], (_NUM_HEADS, _LINE), (0, 1))
            acc = jnp.where(bucket == b, col, acc)
        line_ref[...] = acc

    band_t = jnp.clip(qi - 1, 0, _SEQ // _BQ - 3)
    col0 = pl.multiple_of(band_t * _BQ, _BQ)
    base_b = pl.multiple_of(col0 - qi * _BQ + (_SEQ - _BQ), _BQ)
    for h in range(_NUM_HEADS):
        c15 = line_ref[h, _SEQ - 2 * _BQ]
        c31 = line_ref[h, _SEQ + 2 * _BQ - 1]
        for t in range(_SEQ // _BQ):
            tile_val = jnp.where(t <= qi, c15, c31)
            out_ref[0, h, :, t * _BQ:(t + 1) * _BQ] = (
                jax.lax.broadcast_in_dim(tile_val, (_BQ, _BQ), ()))
        w = line_ref[h, pl.ds(base_b, 4 * _BQ)]
        x = jax.lax.broadcast_in_dim(w, (_BQ, 4 * _BQ), (1,))
        y = pltpu.roll(x, 4 * _BQ - (_BQ - 1), 1, stride=1, stride_axis=0)
        out_ref[0, h, :, pl.ds(col0, 3 * _BQ)] = y[:, :3 * _BQ]


def kernel(q_len, k_len, bidirectional, table):
    del q_len, k_len, bidirectional
    return pl.pallas_call(
        _bias_body,
        grid=(_SEQ // _BQ,),
        in_specs=[pl.BlockSpec((_NUM_HEADS, _NUM_BUCKETS), lambda i: (0, 0))],
        out_specs=pl.BlockSpec(
            (1, _NUM_HEADS, _BQ, _SEQ), lambda i: (0, 0, i, 0)),
        out_shape=jax.ShapeDtypeStruct(
            (1, _NUM_HEADS, _SEQ, _SEQ), jnp.float32),
        scratch_shapes=[pltpu.VMEM((_NUM_HEADS, _LINE), jnp.float32)],
        compiler_params=pltpu.CompilerParams(
            dimension_semantics=("arbitrary",)),
    )(table.T)
